# baseline (speedup 1.0000x reference)
import functools

import jax
import jax.numpy as jnp
from jax import lax
from jax.experimental import pallas as pl
from jax.experimental.pallas import tpu as pltpu
from jax.experimental.pallas import tpu_sc as plsc

VOCAB = 1000000
DIM = 64
SEQ = 200
BATCH = 1024

NC = 2
NS = 16
NW = NC * NS
SEQ_PER_W = BATCH // NW
CHUNK_SEQ = 8
CHUNK_ROWS = CHUNK_SEQ * SEQ
N_CHUNKS = SEQ_PER_W // CHUNK_SEQ
NLANE = 16
DREG = DIM // NLANE


def _sc_body(x_hbm, table_hbm, pos_hbm, out_hbm, idx_v, rows_v, pos_v, sem):
    wid = lax.axis_index("s") * NC + lax.axis_index("c")
    pltpu.sync_copy(pos_hbm, pos_v)

    def chunk_body(i, carry):
        base_row = (wid * SEQ_PER_W + i * CHUNK_SEQ) * SEQ
        pltpu.sync_copy(x_hbm.at[pl.ds(base_row, CHUNK_ROWS)], idx_v)
        pltpu.async_copy(table_hbm.at[idx_v], rows_v, sem).wait()

        def l_body(l, carry_l):
            def s_body(s, carry_s):
                r = s * SEQ + l
                for c in range(DREG):
                    sl = pl.ds(c * NLANE, NLANE)
                    rows_v[r, sl] = rows_v[r, sl] + pos_v[l, sl]
                return carry_s
            return lax.fori_loop(0, CHUNK_SEQ, s_body, carry_l)

        lax.fori_loop(0, SEQ, l_body, 0)
        pltpu.sync_copy(rows_v, out_hbm.at[pl.ds(base_row, CHUNK_ROWS)])
        return carry

    lax.fori_loop(0, N_CHUNKS, chunk_body, 0)


@jax.jit
def kernel(x, table, pos_emb):
    x_flat = x.reshape(-1).astype(jnp.int32)
    mesh = plsc.VectorSubcoreMesh(core_axis_name="c", subcore_axis_name="s")
    run = functools.partial(
        pl.kernel,
        mesh=mesh,
        compiler_params=pltpu.CompilerParams(
            use_tc_tiling_on_sc=False, skip_device_barrier=True
        ),
        out_type=jax.ShapeDtypeStruct((BATCH * SEQ, DIM), jnp.float32),
        scratch_types=[
            pltpu.VMEM((CHUNK_ROWS,), jnp.int32),
            pltpu.VMEM((CHUNK_ROWS, DIM), jnp.float32),
            pltpu.VMEM((SEQ, DIM), jnp.float32),
            pltpu.SemaphoreType.DMA,
        ],
    )(_sc_body)
    out = run(x_flat, table, pos_emb)
    return out.reshape(BATCH, SEQ, DIM)


# speedup vs baseline: 1.1451x; 1.1451x over previous
import functools

import jax
import jax.numpy as jnp
from jax import lax
from jax.experimental import pallas as pl
from jax.experimental.pallas import tpu as pltpu
from jax.experimental.pallas import tpu_sc as plsc

VOCAB = 1000000
DIM = 64
SEQ = 200
BATCH = 1024

NC = 2
NS = 16
NW = NC * NS
SEQ_PER_W = BATCH // NW
CHUNK_SEQ = 4
CHUNK_ROWS = CHUNK_SEQ * SEQ
N_CHUNKS = SEQ_PER_W // CHUNK_SEQ
NLANE = 16
DREG = DIM // NLANE
DPAD = 128


def _sc_body(x_hbm, table_hbm, pos_hbm, out_hbm, idx_v, rows_v, pos_v, sem):
    wid = lax.axis_index("s") * NC + lax.axis_index("c")
    pltpu.sync_copy(pos_hbm, pos_v)

    def chunk_body(i, carry):
        base_row = (wid * SEQ_PER_W + i * CHUNK_SEQ) * SEQ
        pltpu.sync_copy(x_hbm.at[pl.ds(base_row, CHUNK_ROWS)], idx_v)
        pltpu.async_copy(table_hbm.at[idx_v], rows_v, sem).wait()

        def l_body(l, carry_l):
            def s_body(s, carry_s):
                r = s * SEQ + l
                for c in range(DREG):
                    sl = pl.ds(c * NLANE, NLANE)
                    rows_v[r, sl] = rows_v[r, sl] + pos_v[l, sl]
                return carry_s
            return lax.fori_loop(0, CHUNK_SEQ, s_body, carry_l)

        lax.fori_loop(0, SEQ, l_body, 0)
        pltpu.sync_copy(rows_v, out_hbm.at[pl.ds(base_row, CHUNK_ROWS)])
        return carry

    lax.fori_loop(0, N_CHUNKS, chunk_body, 0)


@jax.jit
def kernel(x, table, pos_emb):
    x_flat = x.reshape(-1).astype(jnp.int32)
    tpad = jnp.pad(table, ((0, 0), (0, DPAD - DIM)))
    ppad = jnp.pad(pos_emb, ((0, 0), (0, DPAD - DIM)))
    mesh = plsc.VectorSubcoreMesh(core_axis_name="c", subcore_axis_name="s")
    run = functools.partial(
        pl.kernel,
        mesh=mesh,
        compiler_params=pltpu.CompilerParams(
            use_tc_tiling_on_sc=False, skip_device_barrier=True
        ),
        out_type=jax.ShapeDtypeStruct((BATCH * SEQ, DPAD), jnp.float32),
        scratch_types=[
            pltpu.VMEM((CHUNK_ROWS,), jnp.int32),
            pltpu.VMEM((CHUNK_ROWS, DPAD), jnp.float32),
            pltpu.VMEM((SEQ, DPAD), jnp.float32),
            pltpu.SemaphoreType.DMA,
        ],
    )(_sc_body)
    out = run(x_flat, tpad, ppad)
    return out[:, :DIM].reshape(BATCH, SEQ, DIM)
